# baseline (speedup 1.0000x reference)
import functools

import jax
import jax.numpy as jnp
from jax import lax
from jax.experimental import pallas as pl
from jax.experimental.pallas import tpu as pltpu
from jax.experimental.pallas import tpu_sc as plsc

D_MODEL = 2048


@functools.partial(
    pl.kernel,
    mesh=plsc.VectorSubcoreMesh(core_axis_name="c", subcore_axis_name="s"),
    out_type=jax.ShapeDtypeStruct((D_MODEL,), jnp.float32),
    scratch_types=[
        pltpu.VMEM((1,), jnp.int32),
        pltpu.VMEM((1, D_MODEL), jnp.float32),
        pltpu.SemaphoreType.DMA,
    ],
)
def _sc_row_gather(table_hbm, idx_hbm, out_hbm, idx_v, row_v, sem):
    cid = lax.axis_index("c")
    sid = lax.axis_index("s")

    @pl.when(jnp.logical_and(cid == 0, sid == 0))
    def _():
        pltpu.sync_copy(idx_hbm, idx_v)
        pltpu.async_copy(table_hbm.at[idx_v], row_v, sem).wait()
        pltpu.sync_copy(row_v.at[0], out_hbm)


def kernel(preProcess, xx, state):
    out = _sc_row_gather(preProcess, xx)
    return (out, state)


# speedup vs baseline: 1.1529x; 1.1529x over previous
import functools

import jax
import jax.numpy as jnp
from jax import lax
from jax.experimental import pallas as pl
from jax.experimental.pallas import tpu as pltpu
from jax.experimental.pallas import tpu_sc as plsc

D_MODEL = 2048


@functools.partial(
    pl.kernel,
    mesh=plsc.ScalarSubcoreMesh(axis_name="c", num_cores=1),
    out_type=jax.ShapeDtypeStruct((D_MODEL,), jnp.float32),
    scratch_types=[
        pltpu.SMEM((1,), jnp.int32),
        pltpu.MemorySpace.VMEM_SHARED((1, D_MODEL), jnp.float32),
    ],
)
def _sc_row_gather(table_hbm, idx_hbm, out_hbm, idx_s, row_sp):
    pltpu.sync_copy(idx_hbm, idx_s)
    i = idx_s[0]
    pltpu.sync_copy(table_hbm.at[pl.ds(i, 1)], row_sp)
    pltpu.sync_copy(row_sp.at[0], out_hbm)


def kernel(preProcess, xx, state):
    out = _sc_row_gather(preProcess, xx)
    return (out, state)
